# baseline (speedup 1.0000x reference)
import jax
import jax.numpy as jnp
import numpy as np
from jax import lax
from jax.experimental import pallas as pl
from jax.experimental.pallas import tpu as pltpu
from jax.experimental.pallas import tpu_sc as plsc

_K = 8192
_D = 256
_NTILE = 16
_CHUNK = _K // _NTILE
_BIG_I = np.int32(2**30)


_TC_BLK = 2048
_TC_NCH = _K // _TC_BLK
_TC_NBUF = 2


def _dists_body(z_ref, sym_ref, cb_hbm, o_ref, osym_ref, *scr):
    bufs, sems = scr[:_TC_NBUF], scr[_TC_NBUF:]

    def cp(i):
        return pltpu.make_async_copy(
            cb_hbm.at[pl.ds(i * _TC_BLK, _TC_BLK), :],
            bufs[i % _TC_NBUF], sems[i % _TC_NBUF])

    for i in range(_TC_NBUF):
        cp(i).start()
    z = z_ref[...]
    for i in range(_TC_NCH):
        cp(i).wait()
        d = bufs[i % _TC_NBUF][...] - z
        o_ref[pl.ds(i * _TC_BLK, _TC_BLK)] = jnp.sum(d * d, axis=1)
        if i + _TC_NBUF < _TC_NCH:
            cp(i + _TC_NBUF).start()
    osym_ref[...] = jnp.zeros((16,), jnp.int32) + sym_ref[0]


def _tc_dists(z_flat, codebook, current_sym):
    sym1 = jnp.asarray(current_sym, dtype=jnp.int32).reshape(1)
    return pl.pallas_call(
        _dists_body,
        in_specs=[
            pl.BlockSpec((1, _D), memory_space=pltpu.VMEM),
            pl.BlockSpec(memory_space=pltpu.SMEM),
            pl.BlockSpec(memory_space=pl.ANY),
        ],
        out_specs=[
            pl.BlockSpec(memory_space=pltpu.VMEM),
            pl.BlockSpec(memory_space=pltpu.VMEM),
        ],
        out_shape=[
            jax.ShapeDtypeStruct((_K,), jnp.float32),
            jax.ShapeDtypeStruct((16,), jnp.int32),
        ],
        scratch_shapes=(
            [pltpu.VMEM((_TC_BLK, _D), jnp.float32)] * _TC_NBUF
            + [pltpu.SemaphoreType.DMA] * _TC_NBUF
        ),
    )(z_flat.reshape(1, _D), sym1, codebook)


def _lex_bcast(vv, ii, rrf, rri):
    for s in (8, 4, 2, 1):
        rrf[pl.ds(0, 16)] = vv
        rrf[pl.ds(16, 16)] = vv
        rri[pl.ds(0, 16)] = ii
        rri[pl.ds(16, 16)] = ii
        v2 = rrf[pl.ds(s, 16)]
        i2 = rri[pl.ds(s, 16)]
        take = (v2 < vv) | ((v2 == vv) & (i2 < ii))
        vv = jnp.where(take, v2, vv)
        ii = jnp.where(take, i2, ii)
    return vv, ii


def _min_bcast_f(vv, rrf):
    for s in (8, 4, 2, 1):
        rrf[pl.ds(0, 16)] = vv
        rrf[pl.ds(16, 16)] = vv
        vv = jnp.minimum(vv, rrf[pl.ds(s, 16)])
    return vv


def _min_bcast_i(ii, rri):
    for s in (8, 4, 2, 1):
        rri[pl.ds(0, 16)] = ii
        rri[pl.ds(16, 16)] = ii
        ii = jnp.minimum(ii, rri[pl.ds(s, 16)])
    return ii


def _sc_body(dists_hbm, adj_hbm, sym_hbm, best_hbm, score_hbm,
             dv, sym_v, arow, gd, ob, osc, rrf, rri, lv, li,
             sh_v, sh_i, sem, sem2):
    t = lax.axis_index("s")
    lanes = lax.iota(jnp.int32, 16)
    inf = jnp.float32(jnp.inf)

    cp_sym = pltpu.async_copy(sym_hbm, sym_v, sem)
    pltpu.sync_copy(dists_hbm.at[pl.ds(t * _CHUNK, _CHUNK)], dv)

    v4 = jnp.full((16,), inf, jnp.float32)
    i4 = jnp.zeros((16,), jnp.int32)
    for p in range(4):
        mv = jnp.full((16,), inf, jnp.float32)
        ev = jnp.full((16,), _BIG_I, jnp.int32)
        for j in range(_CHUNK // 16):
            v = dv[pl.ds(j * 16, 16)]
            c = v < mv
            mv = jnp.where(c, v, mv)
            ev = jnp.where(c, lanes + j * 16, ev)
        bv, be = _lex_bcast(mv, ev, rrf, rri)
        v4 = jnp.where(lanes == p, bv, v4)
        i4 = jnp.where(lanes == p, be + t * _CHUNK, i4)
        if p < 3:
            minpos = be[0]
            start = (minpos // 16) * 16
            blk = dv[pl.ds(start, 16)]
            dv[pl.ds(start, 16)] = jnp.where(lanes + start == minpos, inf, blk)

    osc[...] = v4
    ob[...] = i4
    pltpu.sync_copy(osc, sh_v.at[pl.ds(t * 16, 16)])
    pltpu.sync_copy(ob, sh_i.at[pl.ds(t * 16, 16)])
    cp_sym.wait()
    plsc.subcore_barrier()

    @pl.when(t == 0)
    def _():
        symv = sym_v[...]
        sym0 = symv[0]

        cp_row = pltpu.async_copy(adj_hbm.at[sym0], arow, sem2)
        dbase = (sym0 // 16) * 16
        cp_d = pltpu.async_copy(dists_hbm.at[pl.ds(dbase, 16)], gd, sem2)

        pltpu.sync_copy(sh_v, lv)
        pltpu.sync_copy(sh_i, li)

        cvals, cidxv, alanes, abases = [], [], [], []
        for p in range(4):
            mv = jnp.full((16,), inf, jnp.float32)
            mi = jnp.full((16,), _BIG_I, jnp.int32)
            for j in range(16):
                v = lv[pl.ds(j * 16, 16)]
                ix = li[pl.ds(j * 16, 16)]
                take = (v < mv) | ((v == mv) & (ix < mi))
                mv = jnp.where(take, v, mv)
                mi = jnp.where(take, ix, mi)
            bv, bidx = _lex_bcast(mv, mi, rrf, rri)
            cvals.append(bv)
            cidxv.append(bidx)
            ci = bidx[0]
            abase = (ci // 16) * 16
            abases.append(abase)
            alanes.append(ci - abase)
            if p < 3:
                for j in range(16):
                    v = lv[pl.ds(j * 16, 16)]
                    ix = li[pl.ds(j * 16, 16)]
                    lv[pl.ds(j * 16, 16)] = jnp.where(
                        (v == bv) & (ix == bidx), inf, v)

        cp_row.wait()
        cp_d.wait()

        dsel = jnp.where(lanes == sym0 - dbase, gd[...], inf)
        d_sym = _min_bcast_f(dsel, rrf)

        scorev = jnp.where(lanes == 0, d_sym, inf)
        idv = jnp.where(lanes == 0, symv, 0)
        for p in range(4):
            gav = arow[pl.ds(abases[p], 16)]
            asel = jnp.where(lanes == alanes[p], gav, inf)
            a_p = _min_bcast_f(asel, rrf)
            sc = cvals[p] + jnp.float32(0.1) * a_p
            scorev = jnp.where(lanes == p + 1, sc, scorev)
            idv = jnp.where(lanes == p + 1, cidxv[p], idv)

        bs, bl = _lex_bcast(scorev, lanes, rrf, rri)
        bi = _min_bcast_i(jnp.where(lanes == bl, idv, _BIG_I), rri)
        ob[...] = bi
        osc[...] = bs
        cp_b = pltpu.async_copy(ob, best_hbm, sem2)
        cp_s = pltpu.async_copy(osc, score_hbm, sem2)
        cp_b.wait()
        cp_s.wait()


def _sc_select(dists, adjacency, sym16):
    mesh = plsc.VectorSubcoreMesh(
        core_axis_name="c", subcore_axis_name="s",
        num_cores=1, num_subcores=_NTILE)
    f = pl.kernel(
        _sc_body,
        out_type=(
            jax.ShapeDtypeStruct((16,), jnp.int32),
            jax.ShapeDtypeStruct((16,), jnp.float32),
        ),
        mesh=mesh,
        scratch_types=[
            pltpu.VMEM((_CHUNK,), jnp.float32),
            pltpu.VMEM((16,), jnp.int32),
            pltpu.VMEM((_K,), jnp.float32),
            pltpu.VMEM((16,), jnp.float32),
            pltpu.VMEM((16,), jnp.int32),
            pltpu.VMEM((16,), jnp.float32),
            pltpu.VMEM((32,), jnp.float32),
            pltpu.VMEM((32,), jnp.int32),
            pltpu.VMEM((256,), jnp.float32),
            pltpu.VMEM((256,), jnp.int32),
            pltpu.VMEM_SHARED((256,), jnp.float32),
            pltpu.VMEM_SHARED((256,), jnp.int32),
            pltpu.SemaphoreType.DMA,
            pltpu.SemaphoreType.DMA,
        ],
    )
    return f(dists, adjacency, sym16)


def kernel(z_flat, codebook, adjacency, current_sym):
    dists, sym16 = _tc_dists(z_flat, codebook, current_sym)
    best16, score16 = _sc_select(dists, adjacency, sym16)
    return best16[0], score16[0]


# speedup vs baseline: 1.0100x; 1.0100x over previous
import jax
import jax.numpy as jnp
import numpy as np
from jax import lax
from jax.experimental import pallas as pl
from jax.experimental.pallas import tpu as pltpu
from jax.experimental.pallas import tpu_sc as plsc

_K = 8192
_D = 256
_NTILE = 16
_CHUNK = _K // _NTILE
_BIG_I = np.int32(2**30)


_TC_BLK = 1024
_TC_NCH = _K // _TC_BLK
_TC_NBUF = 3


def _dists_body(z_ref, sym_ref, cb_hbm, o_ref, osym_ref, *scr):
    bufs, sems = scr[:_TC_NBUF], scr[_TC_NBUF:]

    def cp(i):
        return pltpu.make_async_copy(
            cb_hbm.at[pl.ds(i * _TC_BLK, _TC_BLK), :],
            bufs[i % _TC_NBUF], sems[i % _TC_NBUF])

    for i in range(_TC_NBUF):
        cp(i).start()
    z = z_ref[...]
    for i in range(_TC_NCH):
        cp(i).wait()
        d = bufs[i % _TC_NBUF][...] - z
        o_ref[pl.ds(i * _TC_BLK, _TC_BLK)] = jnp.sum(d * d, axis=1)
        if i + _TC_NBUF < _TC_NCH:
            cp(i + _TC_NBUF).start()
    osym_ref[...] = jnp.zeros((16,), jnp.int32) + sym_ref[0]


def _tc_dists(z_flat, codebook, current_sym):
    sym1 = jnp.asarray(current_sym, dtype=jnp.int32).reshape(1)
    return pl.pallas_call(
        _dists_body,
        in_specs=[
            pl.BlockSpec((1, _D), memory_space=pltpu.VMEM),
            pl.BlockSpec(memory_space=pltpu.SMEM),
            pl.BlockSpec(memory_space=pl.ANY),
        ],
        out_specs=[
            pl.BlockSpec(memory_space=pltpu.VMEM),
            pl.BlockSpec(memory_space=pltpu.VMEM),
        ],
        out_shape=[
            jax.ShapeDtypeStruct((_K,), jnp.float32),
            jax.ShapeDtypeStruct((16,), jnp.int32),
        ],
        scratch_shapes=(
            [pltpu.VMEM((_TC_BLK, _D), jnp.float32)] * _TC_NBUF
            + [pltpu.SemaphoreType.DMA] * _TC_NBUF
        ),
    )(z_flat.reshape(1, _D), sym1, codebook)


def _lex_bcast(vv, ii, rrf, rri):
    for s in (8, 4, 2, 1):
        rrf[pl.ds(0, 16)] = vv
        rrf[pl.ds(16, 16)] = vv
        rri[pl.ds(0, 16)] = ii
        rri[pl.ds(16, 16)] = ii
        v2 = rrf[pl.ds(s, 16)]
        i2 = rri[pl.ds(s, 16)]
        take = (v2 < vv) | ((v2 == vv) & (i2 < ii))
        vv = jnp.where(take, v2, vv)
        ii = jnp.where(take, i2, ii)
    return vv, ii


def _min_bcast_f(vv, rrf):
    for s in (8, 4, 2, 1):
        rrf[pl.ds(0, 16)] = vv
        rrf[pl.ds(16, 16)] = vv
        vv = jnp.minimum(vv, rrf[pl.ds(s, 16)])
    return vv


def _min_bcast_i(ii, rri):
    for s in (8, 4, 2, 1):
        rri[pl.ds(0, 16)] = ii
        rri[pl.ds(16, 16)] = ii
        ii = jnp.minimum(ii, rri[pl.ds(s, 16)])
    return ii


def _sc_body(dists_hbm, adj_hbm, sym_hbm, best_hbm, score_hbm,
             dv, sym_v, arow, gd, ob, osc, rrf, rri, lv, li,
             sh_v, sh_i, sem, sem2):
    t = lax.axis_index("s")
    lanes = lax.iota(jnp.int32, 16)
    inf = jnp.float32(jnp.inf)

    cp_sym = pltpu.async_copy(sym_hbm, sym_v, sem)
    pltpu.sync_copy(dists_hbm.at[pl.ds(t * _CHUNK, _CHUNK)], dv)

    v4 = jnp.full((16,), inf, jnp.float32)
    i4 = jnp.zeros((16,), jnp.int32)
    for p in range(4):
        mv = jnp.full((16,), inf, jnp.float32)
        ev = jnp.full((16,), _BIG_I, jnp.int32)
        for j in range(_CHUNK // 16):
            v = dv[pl.ds(j * 16, 16)]
            c = v < mv
            mv = jnp.where(c, v, mv)
            ev = jnp.where(c, lanes + j * 16, ev)
        bv, be = _lex_bcast(mv, ev, rrf, rri)
        v4 = jnp.where(lanes == p, bv, v4)
        i4 = jnp.where(lanes == p, be + t * _CHUNK, i4)
        if p < 3:
            minpos = be[0]
            start = (minpos // 16) * 16
            blk = dv[pl.ds(start, 16)]
            dv[pl.ds(start, 16)] = jnp.where(lanes + start == minpos, inf, blk)

    osc[...] = v4
    ob[...] = i4
    pltpu.sync_copy(osc, sh_v.at[pl.ds(t * 16, 16)])
    pltpu.sync_copy(ob, sh_i.at[pl.ds(t * 16, 16)])
    cp_sym.wait()
    plsc.subcore_barrier()

    @pl.when(t == 0)
    def _():
        symv = sym_v[...]
        sym0 = symv[0]

        cp_row = pltpu.async_copy(adj_hbm.at[sym0], arow, sem2)
        dbase = (sym0 // 16) * 16
        cp_d = pltpu.async_copy(dists_hbm.at[pl.ds(dbase, 16)], gd, sem2)

        pltpu.sync_copy(sh_v, lv)
        pltpu.sync_copy(sh_i, li)

        cvals, cidxv, alanes, abases = [], [], [], []
        for p in range(4):
            mv = jnp.full((16,), inf, jnp.float32)
            mi = jnp.full((16,), _BIG_I, jnp.int32)
            for j in range(16):
                v = lv[pl.ds(j * 16, 16)]
                ix = li[pl.ds(j * 16, 16)]
                take = (v < mv) | ((v == mv) & (ix < mi))
                mv = jnp.where(take, v, mv)
                mi = jnp.where(take, ix, mi)
            bv, bidx = _lex_bcast(mv, mi, rrf, rri)
            cvals.append(bv)
            cidxv.append(bidx)
            ci = bidx[0]
            abase = (ci // 16) * 16
            abases.append(abase)
            alanes.append(ci - abase)
            if p < 3:
                for j in range(16):
                    v = lv[pl.ds(j * 16, 16)]
                    ix = li[pl.ds(j * 16, 16)]
                    lv[pl.ds(j * 16, 16)] = jnp.where(
                        (v == bv) & (ix == bidx), inf, v)

        cp_row.wait()
        cp_d.wait()

        dsel = jnp.where(lanes == sym0 - dbase, gd[...], inf)
        d_sym = _min_bcast_f(dsel, rrf)

        scorev = jnp.where(lanes == 0, d_sym, inf)
        idv = jnp.where(lanes == 0, symv, 0)
        for p in range(4):
            gav = arow[pl.ds(abases[p], 16)]
            asel = jnp.where(lanes == alanes[p], gav, inf)
            a_p = _min_bcast_f(asel, rrf)
            sc = cvals[p] + jnp.float32(0.1) * a_p
            scorev = jnp.where(lanes == p + 1, sc, scorev)
            idv = jnp.where(lanes == p + 1, cidxv[p], idv)

        bs, bl = _lex_bcast(scorev, lanes, rrf, rri)
        bi = _min_bcast_i(jnp.where(lanes == bl, idv, _BIG_I), rri)
        ob[...] = bi
        osc[...] = bs
        cp_b = pltpu.async_copy(ob, best_hbm, sem2)
        cp_s = pltpu.async_copy(osc, score_hbm, sem2)
        cp_b.wait()
        cp_s.wait()


def _sc_select(dists, adjacency, sym16):
    mesh = plsc.VectorSubcoreMesh(
        core_axis_name="c", subcore_axis_name="s",
        num_cores=1, num_subcores=_NTILE)
    f = pl.kernel(
        _sc_body,
        out_type=(
            jax.ShapeDtypeStruct((16,), jnp.int32),
            jax.ShapeDtypeStruct((16,), jnp.float32),
        ),
        mesh=mesh,
        scratch_types=[
            pltpu.VMEM((_CHUNK,), jnp.float32),
            pltpu.VMEM((16,), jnp.int32),
            pltpu.VMEM((_K,), jnp.float32),
            pltpu.VMEM((16,), jnp.float32),
            pltpu.VMEM((16,), jnp.int32),
            pltpu.VMEM((16,), jnp.float32),
            pltpu.VMEM((32,), jnp.float32),
            pltpu.VMEM((32,), jnp.int32),
            pltpu.VMEM((256,), jnp.float32),
            pltpu.VMEM((256,), jnp.int32),
            pltpu.VMEM_SHARED((256,), jnp.float32),
            pltpu.VMEM_SHARED((256,), jnp.int32),
            pltpu.SemaphoreType.DMA,
            pltpu.SemaphoreType.DMA,
        ],
    )
    return f(dists, adjacency, sym16)


def kernel(z_flat, codebook, adjacency, current_sym):
    dists, sym16 = _tc_dists(z_flat, codebook, current_sym)
    best16, score16 = _sc_select(dists, adjacency, sym16)
    return best16[0], score16[0]
